# baseline (speedup 1.0000x reference)
import functools

import jax
import jax.numpy as jnp
from jax import lax
from jax.experimental import pallas as pl
from jax.experimental.pallas import tpu as pltpu
from jax.experimental.pallas import tpu_sc as plsc

_REPEAT = 16
_LANES = 16
_NC = 2
_NS = 16
_CH = 2048
_OCH = _CH * _REPEAT


def _sc_body(in_hbm, out_hbm, in_v, out_v):
    b, u = in_hbm.shape
    rows_per_w = b // (_NC * _NS)
    n_ch = u // _CH
    wid = lax.axis_index("s") * _NC + lax.axis_index("c")
    row0 = wid * rows_per_w

    def row_body(r, carry):
        row = row0 + r

        def ch_body(c, carry):
            pltpu.sync_copy(in_hbm.at[row, pl.ds(c * _CH, _CH)], in_v)

            def k_body(k, carry):
                x = in_v[pl.ds(k * _LANES, _LANES)]
                base = k * _LANES * _LANES
                for j in range(_LANES):
                    y = x.at[jnp.full((_LANES,), j, dtype=jnp.int32)].get(
                        mode="promise_in_bounds")
                    out_v[pl.ds(base + j * _LANES, _LANES)] = y
                return carry

            lax.fori_loop(0, _CH // _LANES, k_body, 0, unroll=2)
            pltpu.sync_copy(out_v, out_hbm.at[row, pl.ds(c * _OCH, _OCH)])
            return carry

        return lax.fori_loop(0, n_ch, ch_body, carry)

    lax.fori_loop(0, rows_per_w, row_body, 0)


def kernel(inputs):
    b, u = inputs.shape
    mesh = plsc.VectorSubcoreMesh(core_axis_name="c", subcore_axis_name="s")
    f = functools.partial(
        pl.kernel,
        mesh=mesh,
        out_type=jax.ShapeDtypeStruct((b, u * _REPEAT), inputs.dtype),
        scratch_types=[
            pltpu.VMEM((_CH,), jnp.float32),
            pltpu.VMEM((_OCH,), jnp.float32),
        ],
    )(_sc_body)
    return f(inputs)


# speedup vs baseline: 1.8247x; 1.8247x over previous
import functools

import jax
import jax.numpy as jnp
from jax import lax
from jax.experimental import pallas as pl
from jax.experimental.pallas import tpu as pltpu
from jax.experimental.pallas import tpu_sc as plsc

_REPEAT = 16
_LANES = 16
_NC = 2
_NS = 16
_QUARTS = 8
_NBUF = 2


def _expand(in_v, out_buf, q, n_k):
    def k_body(k, carry):
        x = in_v[pl.ds((q * n_k + k) * _LANES, _LANES)]
        base = k * _LANES * _REPEAT
        for j in range(_REPEAT):
            y = x.at[jnp.full((_LANES,), j, dtype=jnp.int32)].get(
                mode="promise_in_bounds")
            out_buf[pl.ds(base + j * _LANES, _LANES)] = y
        return carry

    lax.fori_loop(0, n_k, k_body, 0, unroll=2)


def _sc_body(in_hbm, out_hbm, in_v, out_v0, out_v1, so0, so1):
    b, u = in_hbm.shape
    rows_per_w = b // (_NC * _NS)
    och = (u // _QUARTS) * _REPEAT
    n_k = u // (_QUARTS * _LANES)
    outs = (out_v0, out_v1)
    sems = (so0, so1)
    wid = lax.axis_index("s") * _NC + lax.axis_index("c")
    row0 = wid * rows_per_w

    def row_body(r, carry):
        row = row0 + r
        pltpu.sync_copy(in_hbm.at[row, :], in_v)
        for q in range(_QUARTS):
            p = q % _NBUF
            g = r * _QUARTS + q
            dst = out_hbm.at[row, pl.ds(q * och, och)]

            @pl.when(g >= _NBUF)
            def _drain():
                pltpu.make_async_copy(outs[p], dst, sems[p]).wait()

            _expand(in_v, outs[p], q, n_k)
            pltpu.async_copy(outs[p], dst, sems[p])
        return carry

    lax.fori_loop(0, rows_per_w, row_body, 0)
    last = row0 + rows_per_w - 1
    for p in range(_NBUF):
        q = _QUARTS - _NBUF + p
        dst = out_hbm.at[last, pl.ds(q * och, och)]
        pltpu.make_async_copy(outs[p], dst, sems[p]).wait()


def kernel(inputs):
    b, u = inputs.shape
    och = (u // _QUARTS) * _REPEAT
    mesh = plsc.VectorSubcoreMesh(core_axis_name="c", subcore_axis_name="s")
    f = functools.partial(
        pl.kernel,
        mesh=mesh,
        out_type=jax.ShapeDtypeStruct((b, u * _REPEAT), inputs.dtype),
        scratch_types=[
            pltpu.VMEM((u,), jnp.float32),
            pltpu.VMEM((och,), jnp.float32),
            pltpu.VMEM((och,), jnp.float32),
            pltpu.SemaphoreType.DMA,
            pltpu.SemaphoreType.DMA,
        ],
    )(_sc_body)
    return f(inputs)


# speedup vs baseline: 1.9512x; 1.0693x over previous
import functools

import jax
import jax.numpy as jnp
from jax import lax
from jax.experimental import pallas as pl
from jax.experimental.pallas import tpu as pltpu
from jax.experimental.pallas import tpu_sc as plsc

_REPEAT = 16
_LANES = 16
_NC = 2
_NS = 16
_QUARTS = 8
_NBUF = 2


def _expand(in_v, out_buf, q, n_k):
    def k_body(k, carry):
        x = in_v[pl.ds((q * n_k + k) * _LANES, _LANES)]
        base = k * _LANES * _REPEAT
        for j in range(_REPEAT):
            y = x.at[jnp.full((_LANES,), j, dtype=jnp.int32)].get(
                mode="promise_in_bounds")
            out_buf[pl.ds(base + j * _LANES, _LANES)] = y
        return carry

    lax.fori_loop(0, n_k, k_body, 0, unroll=2)


def _sc_body(in_hbm, out_hbm, in_v0, in_v1, out_v0, out_v1,
             si0, si1, so0, so1):
    b, u = in_hbm.shape
    rows_per_w = b // (_NC * _NS)
    och = (u // _QUARTS) * _REPEAT
    n_k = u // (_QUARTS * _LANES)
    ins = (in_v0, in_v1)
    isems = (si0, si1)
    outs = (out_v0, out_v1)
    sems = (so0, so1)
    wid = lax.axis_index("s") * _NC + lax.axis_index("c")
    row0 = wid * rows_per_w

    def do_row(row, in_buf, in_sem, first_row):
        pltpu.make_async_copy(in_hbm.at[row, :], in_buf, in_sem).wait()
        for q in range(_QUARTS):
            p = q % _NBUF
            dst = out_hbm.at[row, pl.ds(q * och, och)]
            if first_row and q < _NBUF:
                pass
            else:
                pltpu.make_async_copy(outs[p], dst, sems[p]).wait()
            _expand(in_buf, outs[p], q, n_k)
            pltpu.async_copy(outs[p], dst, sems[p])

    pltpu.async_copy(in_hbm.at[row0, :], ins[0], isems[0])

    def pair_body(h, carry):
        for s in range(2):
            r = h * 2 + s
            row = row0 + r

            @pl.when(r + 1 < rows_per_w)
            def _prefetch():
                pltpu.async_copy(in_hbm.at[row + 1, :],
                                 ins[1 - s], isems[1 - s])

            @pl.when(r == 0)
            def _first():
                do_row(row, ins[s], isems[s], True)

            @pl.when(r > 0)
            def _rest():
                do_row(row, ins[s], isems[s], False)
        return carry

    lax.fori_loop(0, rows_per_w // 2, pair_body, 0)
    last = row0 + rows_per_w - 1
    for p in range(_NBUF):
        q = _QUARTS - _NBUF + p
        dst = out_hbm.at[last, pl.ds(q * och, och)]
        pltpu.make_async_copy(outs[p], dst, sems[p]).wait()


def kernel(inputs):
    b, u = inputs.shape
    och = (u // _QUARTS) * _REPEAT
    mesh = plsc.VectorSubcoreMesh(core_axis_name="c", subcore_axis_name="s")
    f = functools.partial(
        pl.kernel,
        mesh=mesh,
        out_type=jax.ShapeDtypeStruct((b, u * _REPEAT), inputs.dtype),
        scratch_types=[
            pltpu.VMEM((u,), jnp.float32),
            pltpu.VMEM((u,), jnp.float32),
            pltpu.VMEM((och,), jnp.float32),
            pltpu.VMEM((och,), jnp.float32),
            pltpu.SemaphoreType.DMA,
            pltpu.SemaphoreType.DMA,
            pltpu.SemaphoreType.DMA,
            pltpu.SemaphoreType.DMA,
        ],
    )(_sc_body)
    return f(inputs)
